# baseline (speedup 1.0000x reference)
import jax
import jax.numpy as jnp
from jax import lax
from jax.experimental import pallas as pl
from jax.experimental.pallas import tpu as pltpu

_N = 8192
_M = 4096

_SURFACE_W = 1.0
_HUBER_DELTA = 0.01
_NORMAL_W = 0.5
_CAUCHY_SIGMA = 0.05
_GRAD_W = 0.1
_GRAD_SIGMA = 0.02
_OPACITY_W = 0.01
_SCALE_W = 0.001
_K = 6

_RT = 512
_RM = 512

_IMAX = 2147483647


def _acc_lanes(out_ref, i, *vals):
    lane = lax.broadcasted_iota(jnp.int32, (1, 128), 1)
    row = jnp.zeros((1, 128), jnp.float32)
    for j, v in enumerate(vals):
        row = row + jnp.where(lane == j, v, 0.0)

    @pl.when(i == 0)
    def _():
        out_ref[...] = jnp.zeros_like(out_ref)

    out_ref[...] += row


def _packed_d2(p8, pos_t8_ref):
    pos_t8 = pos_t8_ref[...]
    g = jnp.dot(p8, pos_t8, preferred_element_type=jnp.float32)
    a2 = jnp.sum(p8 * p8, axis=1, keepdims=True)
    gx = pos_t8[0:1, :]
    gy = pos_t8[1:2, :]
    gz = pos_t8[2:3, :]
    b2 = gx * gx + gy * gy + gz * gz
    d2 = jnp.maximum((a2 + b2) - 2.0 * g, 0.0)
    iota = lax.broadcasted_iota(jnp.int32, d2.shape, 1)
    bits = lax.bitcast_convert_type(d2, jnp.int32)
    packed = (bits & jnp.int32(-4096)) | iota
    return d2, packed, iota


def _surface_kernel(tp8_ref, tn_ref, pos_t8_ref, table_ref, out_ref):
    i = pl.program_id(0)
    tp8 = tp8_ref[...]
    d2, packed, _ = _packed_d2(tp8, pos_t8_ref)

    mnp = jnp.min(packed, axis=1, keepdims=True)
    sel = packed == mnp
    mn2 = jnp.min(d2, axis=1)
    nearest_dist = jnp.sqrt(jnp.maximum(mn2, 1e-24))

    onehot = sel.astype(jnp.float32)
    g = jnp.dot(onehot, table_ref[...], preferred_element_type=jnp.float32,
                precision=lax.Precision.HIGHEST)
    nearest_scales = g[:, 0:3]
    near_pos = g[:, 3:6]
    near_rot = g[:, 6:10]

    adaptive = jnp.mean(jnp.exp(nearest_scales), axis=1)
    nd = nearest_dist / (adaptive + 1e-8)
    ax = jnp.abs(nd)
    huber = jnp.where(ax <= _HUBER_DELTA, 0.5 * nd * nd,
                      _HUBER_DELTA * (ax - 0.5 * _HUBER_DELTA))
    surf_sum = jnp.sum(huber)

    qn = near_rot / jnp.maximum(
        jnp.sqrt(jnp.sum(near_rot * near_rot, axis=1, keepdims=True)), 1e-12)
    w, x, y, z = qn[:, 0:1], qn[:, 1:2], qn[:, 2:3], qn[:, 3:4]
    nrm = jnp.concatenate(
        [2 * (x * z + w * y), 2 * (y * z - w * x), 1 - 2 * (x * x + y * y)],
        axis=1)
    tp = tp8[:, 0:3]
    to_p = tp - near_pos
    dp = jnp.sum(nrm * to_p, axis=1, keepdims=True)
    nrm = jnp.where(dp < 0, -nrm, nrm)
    nrm = nrm / jnp.maximum(
        jnp.sqrt(jnp.sum(nrm * nrm, axis=1, keepdims=True)), 1e-12)
    dot = jnp.sum(tn_ref[...] * nrm, axis=1)
    ae = 1.0 - jnp.abs(dot)
    sig2 = _CAUCHY_SIGMA * _CAUCHY_SIGMA
    cauchy = -jnp.log(sig2 / (sig2 + ae * ae) + 1e-8)
    normal_sum = jnp.sum(cauchy)

    _acc_lanes(out_ref, i, surf_sum, normal_sum)


def _knn_kernel(p8_ref, pos_t8_ref, pos_full_ref, op_ref, sc_ref, out_ref):
    i = pl.program_id(0)
    p8 = p8_ref[...]
    _, packed, iota = _packed_d2(p8, pos_t8_ref)

    row_g = i * _RM + lax.broadcasted_iota(jnp.int32, packed.shape, 0)
    packed = jnp.where(iota == row_g, jnp.int32(_IMAX), packed)

    px, py, pz = p8[:, 0:1], p8[:, 1:2], p8[:, 2:3]
    nbx, nby, nbz = [], [], []
    for _ in range(_K):
        mnp = jnp.min(packed, axis=1, keepdims=True)
        sel = packed == mnp
        onehot = sel.astype(jnp.float32)
        nb = jnp.dot(onehot, pos_full_ref[...],
                     preferred_element_type=jnp.float32,
                     precision=lax.Precision.HIGHEST)
        nbx.append(nb[:, 0:1])
        nby.append(nb[:, 1:2])
        nbz.append(nb[:, 2:3])
        packed = jnp.where(sel, jnp.int32(_IMAX), packed)
    nbx = jnp.concatenate(nbx, axis=1)
    nby = jnp.concatenate(nby, axis=1)
    nbz = jnp.concatenate(nbz, axis=1)

    vx = nbx - px
    vy = nby - py
    vz = nbz - pz
    vn = jnp.maximum(jnp.sqrt(vx * vx + vy * vy + vz * vz), 1e-12)
    nvx = vx / vn
    nvy = vy / vn
    nvz = vz / vn

    grad_sum = jnp.float32(0.0)
    for k in range(_K):
        g = (nvx * nvx[:, k:k + 1] + nvy * nvy[:, k:k + 1]
             + nvz * nvz[:, k:k + 1])
        g = jnp.clip(g, -1.0 + 1e-7, 1.0 - 1e-7)
        ang = jnp.arctan2(jnp.sqrt(jnp.maximum(1.0 - g * g, 0.0)), g)
        grad_sum += jnp.sum(jnp.exp(-ang / _GRAD_SIGMA))

    op = op_ref[...]
    op_sum = jnp.sum(op * (1.0 - op))
    scale_sum = jnp.sum(jnp.exp(sc_ref[...]))

    _acc_lanes(out_ref, i, grad_sum, op_sum, scale_sum)


def kernel(tactile_points, tactile_normals, positions, scales, rotations,
           opacity):
    tp8 = jnp.pad(tactile_points, ((0, 0), (0, 5)))
    p8 = jnp.pad(positions, ((0, 0), (0, 5)))
    pos_t8 = p8.T
    table = jnp.concatenate([scales, positions, rotations], axis=1)

    out_a = pl.pallas_call(
        _surface_kernel,
        grid=(_N // _RT,),
        in_specs=[
            pl.BlockSpec((_RT, 8), lambda i: (i, 0)),
            pl.BlockSpec((_RT, 3), lambda i: (i, 0)),
            pl.BlockSpec((8, _M), lambda i: (0, 0)),
            pl.BlockSpec((_M, 10), lambda i: (0, 0)),
        ],
        out_specs=pl.BlockSpec((1, 128), lambda i: (0, 0)),
        out_shape=jax.ShapeDtypeStruct((1, 128), jnp.float32),
        compiler_params=pltpu.CompilerParams(
            dimension_semantics=("arbitrary",)),
    )(tp8, tactile_normals, pos_t8, table)

    out_b = pl.pallas_call(
        _knn_kernel,
        grid=(_M // _RM,),
        in_specs=[
            pl.BlockSpec((_RM, 8), lambda i: (i, 0)),
            pl.BlockSpec((8, _M), lambda i: (0, 0)),
            pl.BlockSpec((_M, 3), lambda i: (0, 0)),
            pl.BlockSpec((_RM, 1), lambda i: (i, 0)),
            pl.BlockSpec((_RM, 3), lambda i: (i, 0)),
        ],
        out_specs=pl.BlockSpec((1, 128), lambda i: (0, 0)),
        out_shape=jax.ShapeDtypeStruct((1, 128), jnp.float32),
        compiler_params=pltpu.CompilerParams(
            dimension_semantics=("arbitrary",)),
    )(p8, pos_t8, positions, opacity, scales)

    surface_loss = out_a[0, 0] / _N
    normal_loss = out_a[0, 1] / _N
    gradient_loss = out_b[0, 0] / (_M * _K * _K)
    opacity_reg = out_b[0, 1] / _M
    scale_reg = out_b[0, 2] / (_M * 3)

    return (_SURFACE_W * surface_loss
            + _NORMAL_W * normal_loss
            + _GRAD_W * gradient_loss
            + _OPACITY_W * opacity_reg
            + _SCALE_W * scale_reg)


# speedup vs baseline: 1.0780x; 1.0780x over previous
import jax
import jax.numpy as jnp
from jax import lax
from jax.experimental import pallas as pl
from jax.experimental.pallas import tpu as pltpu

_N = 8192
_M = 4096

_SURFACE_W = 1.0
_HUBER_DELTA = 0.01
_NORMAL_W = 0.5
_CAUCHY_SIGMA = 0.05
_GRAD_W = 0.1
_GRAD_SIGMA = 0.02
_OPACITY_W = 0.01
_SCALE_W = 0.001
_K = 6

_RT = 256
_RM = 256

_IMAX = 2147483647


def _acc_lanes(out_ref, i, *vals):
    lane = lax.broadcasted_iota(jnp.int32, (1, 128), 1)
    row = jnp.zeros((1, 128), jnp.float32)
    for j, v in enumerate(vals):
        row = row + jnp.where(lane == j, v, 0.0)

    @pl.when(i == 0)
    def _():
        out_ref[...] = jnp.zeros_like(out_ref)

    out_ref[...] += row


def _diff_d2(p, pos_t_ref):
    px, py, pz = p[:, 0:1], p[:, 1:2], p[:, 2:3]
    gx = pos_t_ref[0:1, :]
    gy = pos_t_ref[1:2, :]
    gz = pos_t_ref[2:3, :]
    dx = px - gx
    dy = py - gy
    dz = pz - gz
    return dx * dx + dy * dy + dz * dz


def _surface_kernel(tp_ref, tn_ref, pos_t_ref, table_ref, out_ref):
    i = pl.program_id(0)
    tp = tp_ref[...]
    d2 = _diff_d2(tp, pos_t_ref)

    mn2 = jnp.min(d2, axis=1)
    am = jnp.argmin(d2, axis=1)
    nearest_dist = jnp.sqrt(jnp.maximum(mn2, 1e-24))

    iota = lax.broadcasted_iota(jnp.int32, d2.shape, 1)
    onehot = (iota == am[:, None]).astype(jnp.float32)
    g = jnp.dot(onehot, table_ref[...], preferred_element_type=jnp.float32,
                precision=lax.Precision.HIGHEST)
    nearest_scales = g[:, 0:3]
    near_pos = g[:, 3:6]
    near_rot = g[:, 6:10]

    adaptive = jnp.mean(jnp.exp(nearest_scales), axis=1)
    nd = nearest_dist / (adaptive + 1e-8)
    ax = jnp.abs(nd)
    huber = jnp.where(ax <= _HUBER_DELTA, 0.5 * nd * nd,
                      _HUBER_DELTA * (ax - 0.5 * _HUBER_DELTA))
    surf_sum = jnp.sum(huber)

    qn = near_rot / jnp.maximum(
        jnp.sqrt(jnp.sum(near_rot * near_rot, axis=1, keepdims=True)), 1e-12)
    w, x, y, z = qn[:, 0:1], qn[:, 1:2], qn[:, 2:3], qn[:, 3:4]
    nrm = jnp.concatenate(
        [2 * (x * z + w * y), 2 * (y * z - w * x), 1 - 2 * (x * x + y * y)],
        axis=1)
    to_p = tp - near_pos
    dp = jnp.sum(nrm * to_p, axis=1, keepdims=True)
    nrm = jnp.where(dp < 0, -nrm, nrm)
    nrm = nrm / jnp.maximum(
        jnp.sqrt(jnp.sum(nrm * nrm, axis=1, keepdims=True)), 1e-12)
    dot = jnp.sum(tn_ref[...] * nrm, axis=1)
    ae = 1.0 - jnp.abs(dot)
    sig2 = _CAUCHY_SIGMA * _CAUCHY_SIGMA
    cauchy = -jnp.log(sig2 / (sig2 + ae * ae) + 1e-8)
    normal_sum = jnp.sum(cauchy)

    _acc_lanes(out_ref, i, surf_sum, normal_sum)


def _knn_kernel(pos_ref, pos_t_ref, pos_full_ref, op_ref, sc_ref, out_ref):
    i = pl.program_id(0)
    p = pos_ref[...]
    d2 = _diff_d2(p, pos_t_ref)

    iota = lax.broadcasted_iota(jnp.int32, d2.shape, 1)
    bits = lax.bitcast_convert_type(d2, jnp.int32)
    packed = (bits & jnp.int32(-4096)) | iota
    row_g = i * _RM + lax.broadcasted_iota(jnp.int32, d2.shape, 0)
    packed = jnp.where(iota == row_g, jnp.int32(_IMAX), packed)

    px, py, pz = p[:, 0:1], p[:, 1:2], p[:, 2:3]
    nbx, nby, nbz = [], [], []
    for _ in range(_K):
        mnp = jnp.min(packed, axis=1, keepdims=True)
        sel = packed == mnp
        onehot = sel.astype(jnp.float32)
        nb = jnp.dot(onehot, pos_full_ref[...],
                     preferred_element_type=jnp.float32,
                     precision=lax.Precision.HIGHEST)
        nbx.append(nb[:, 0:1])
        nby.append(nb[:, 1:2])
        nbz.append(nb[:, 2:3])
        packed = jnp.where(sel, jnp.int32(_IMAX), packed)
    nbx = jnp.concatenate(nbx, axis=1)
    nby = jnp.concatenate(nby, axis=1)
    nbz = jnp.concatenate(nbz, axis=1)

    vx = nbx - px
    vy = nby - py
    vz = nbz - pz
    vn = jnp.maximum(jnp.sqrt(vx * vx + vy * vy + vz * vz), 1e-12)
    nvx = vx / vn
    nvy = vy / vn
    nvz = vz / vn

    grad_sum = jnp.float32(0.0)
    for k in range(_K):
        g = (nvx * nvx[:, k:k + 1] + nvy * nvy[:, k:k + 1]
             + nvz * nvz[:, k:k + 1])
        g = jnp.clip(g, -1.0 + 1e-7, 1.0 - 1e-7)
        ang = jnp.arctan2(jnp.sqrt(jnp.maximum(1.0 - g * g, 0.0)), g)
        grad_sum += jnp.sum(jnp.exp(-ang / _GRAD_SIGMA))

    op = op_ref[...]
    op_sum = jnp.sum(op * (1.0 - op))
    scale_sum = jnp.sum(jnp.exp(sc_ref[...]))

    _acc_lanes(out_ref, i, grad_sum, op_sum, scale_sum)


def kernel(tactile_points, tactile_normals, positions, scales, rotations,
           opacity):
    pos_t = positions.T
    table = jnp.concatenate([scales, positions, rotations], axis=1)

    out_a = pl.pallas_call(
        _surface_kernel,
        grid=(_N // _RT,),
        in_specs=[
            pl.BlockSpec((_RT, 3), lambda i: (i, 0)),
            pl.BlockSpec((_RT, 3), lambda i: (i, 0)),
            pl.BlockSpec((3, _M), lambda i: (0, 0)),
            pl.BlockSpec((_M, 10), lambda i: (0, 0)),
        ],
        out_specs=pl.BlockSpec((1, 128), lambda i: (0, 0)),
        out_shape=jax.ShapeDtypeStruct((1, 128), jnp.float32),
        compiler_params=pltpu.CompilerParams(
            dimension_semantics=("arbitrary",)),
    )(tactile_points, tactile_normals, pos_t, table)

    out_b = pl.pallas_call(
        _knn_kernel,
        grid=(_M // _RM,),
        in_specs=[
            pl.BlockSpec((_RM, 3), lambda i: (i, 0)),
            pl.BlockSpec((3, _M), lambda i: (0, 0)),
            pl.BlockSpec((_M, 3), lambda i: (0, 0)),
            pl.BlockSpec((_RM, 1), lambda i: (i, 0)),
            pl.BlockSpec((_RM, 3), lambda i: (i, 0)),
        ],
        out_specs=pl.BlockSpec((1, 128), lambda i: (0, 0)),
        out_shape=jax.ShapeDtypeStruct((1, 128), jnp.float32),
        compiler_params=pltpu.CompilerParams(
            dimension_semantics=("arbitrary",)),
    )(positions, pos_t, positions, opacity, scales)

    surface_loss = out_a[0, 0] / _N
    normal_loss = out_a[0, 1] / _N
    gradient_loss = out_b[0, 0] / (_M * _K * _K)
    opacity_reg = out_b[0, 1] / _M
    scale_reg = out_b[0, 2] / (_M * 3)

    return (_SURFACE_W * surface_loss
            + _NORMAL_W * normal_loss
            + _GRAD_W * gradient_loss
            + _OPACITY_W * opacity_reg
            + _SCALE_W * scale_reg)


# speedup vs baseline: 2.2799x; 2.1149x over previous
import jax
import jax.numpy as jnp
from jax import lax
from jax.experimental import pallas as pl
from jax.experimental.pallas import tpu as pltpu

_N = 8192
_M = 4096

_SURFACE_W = 1.0
_HUBER_DELTA = 0.01
_NORMAL_W = 0.5
_CAUCHY_SIGMA = 0.05
_GRAD_W = 0.1
_GRAD_SIGMA = 0.02
_OPACITY_W = 0.01
_SCALE_W = 0.001
_K = 6

_RT = 256
_RM = 256

_IMAX = 2147483647


def _acc_lanes(out_ref, i, *vals):
    lane = lax.broadcasted_iota(jnp.int32, (1, 128), 1)
    row = jnp.zeros((1, 128), jnp.float32)
    for j, v in enumerate(vals):
        row = row + jnp.where(lane == j, v, 0.0)

    @pl.when(i == 0)
    def _():
        out_ref[...] = jnp.zeros_like(out_ref)

    out_ref[...] += row


def _diff_d2(p, pos_t_ref):
    px, py, pz = p[:, 0:1], p[:, 1:2], p[:, 2:3]
    gx = pos_t_ref[0:1, :]
    gy = pos_t_ref[1:2, :]
    gz = pos_t_ref[2:3, :]
    dx = px - gx
    dy = py - gy
    dz = pz - gz
    return dx * dx + dy * dy + dz * dz


def _surface_kernel(tp_ref, tn_ref, pos_t_ref, table_ref, out_ref):
    i = pl.program_id(0)
    tp = tp_ref[...]
    d2 = _diff_d2(tp, pos_t_ref)

    mn2 = jnp.min(d2, axis=1)
    am = jnp.argmin(d2, axis=1)
    nearest_dist = jnp.sqrt(jnp.maximum(mn2, 1e-24))

    iota = lax.broadcasted_iota(jnp.int32, d2.shape, 1)
    onehot = (iota == am[:, None]).astype(jnp.float32)
    ghl = jnp.dot(onehot, table_ref[...],
                  preferred_element_type=jnp.float32)
    g = ghl[:, 0:10] + ghl[:, 10:20]
    nearest_scales = g[:, 0:3]
    near_pos = g[:, 3:6]
    near_rot = g[:, 6:10]

    adaptive = jnp.mean(jnp.exp(nearest_scales), axis=1)
    nd = nearest_dist / (adaptive + 1e-8)
    ax = jnp.abs(nd)
    huber = jnp.where(ax <= _HUBER_DELTA, 0.5 * nd * nd,
                      _HUBER_DELTA * (ax - 0.5 * _HUBER_DELTA))
    surf_sum = jnp.sum(huber)

    qn = near_rot / jnp.maximum(
        jnp.sqrt(jnp.sum(near_rot * near_rot, axis=1, keepdims=True)), 1e-12)
    w, x, y, z = qn[:, 0:1], qn[:, 1:2], qn[:, 2:3], qn[:, 3:4]
    nrm = jnp.concatenate(
        [2 * (x * z + w * y), 2 * (y * z - w * x), 1 - 2 * (x * x + y * y)],
        axis=1)
    to_p = tp - near_pos
    dp = jnp.sum(nrm * to_p, axis=1, keepdims=True)
    nrm = jnp.where(dp < 0, -nrm, nrm)
    nrm = nrm / jnp.maximum(
        jnp.sqrt(jnp.sum(nrm * nrm, axis=1, keepdims=True)), 1e-12)
    dot = jnp.sum(tn_ref[...] * nrm, axis=1)
    ae = 1.0 - jnp.abs(dot)
    sig2 = _CAUCHY_SIGMA * _CAUCHY_SIGMA
    cauchy = -jnp.log(sig2 / (sig2 + ae * ae) + 1e-8)
    normal_sum = jnp.sum(cauchy)

    _acc_lanes(out_ref, i, surf_sum, normal_sum)


def _knn_kernel(pos_ref, pos_t_ref, pos_full_ref, op_ref, sc_ref, out_ref):
    i = pl.program_id(0)
    p = pos_ref[...]
    d2 = _diff_d2(p, pos_t_ref)

    iota = lax.broadcasted_iota(jnp.int32, d2.shape, 1)
    bits = lax.bitcast_convert_type(d2, jnp.int32)
    packed = (bits & jnp.int32(-4096)) | iota
    row_g = i * _RM + lax.broadcasted_iota(jnp.int32, d2.shape, 0)
    packed = jnp.where(iota == row_g, jnp.int32(_IMAX), packed)

    px, py, pz = p[:, 0:1], p[:, 1:2], p[:, 2:3]
    nbx, nby, nbz = [], [], []
    for _ in range(_K):
        mnp = jnp.min(packed, axis=1, keepdims=True)
        sel = packed == mnp
        onehot = sel.astype(jnp.float32)
        nbhl = jnp.dot(onehot, pos_full_ref[...],
                       preferred_element_type=jnp.float32)
        nb = nbhl[:, 0:3] + nbhl[:, 3:6]
        nbx.append(nb[:, 0:1])
        nby.append(nb[:, 1:2])
        nbz.append(nb[:, 2:3])
        packed = jnp.where(sel, jnp.int32(_IMAX), packed)
    nbx = jnp.concatenate(nbx, axis=1)
    nby = jnp.concatenate(nby, axis=1)
    nbz = jnp.concatenate(nbz, axis=1)

    vx = nbx - px
    vy = nby - py
    vz = nbz - pz
    vn = jnp.maximum(jnp.sqrt(vx * vx + vy * vy + vz * vz), 1e-12)
    nvx = vx / vn
    nvy = vy / vn
    nvz = vz / vn

    grad_sum = jnp.float32(0.0)
    for k in range(_K):
        g = (nvx * nvx[:, k:k + 1] + nvy * nvy[:, k:k + 1]
             + nvz * nvz[:, k:k + 1])
        g = jnp.clip(g, -1.0 + 1e-7, 1.0 - 1e-7)
        ang = jnp.arctan2(jnp.sqrt(jnp.maximum(1.0 - g * g, 0.0)), g)
        grad_sum += jnp.sum(jnp.exp(-ang / _GRAD_SIGMA))

    op = op_ref[...]
    op_sum = jnp.sum(op * (1.0 - op))
    scale_sum = jnp.sum(jnp.exp(sc_ref[...]))

    _acc_lanes(out_ref, i, grad_sum, op_sum, scale_sum)


def kernel(tactile_points, tactile_normals, positions, scales, rotations,
           opacity):
    pos_t = positions.T
    table = jnp.concatenate([scales, positions, rotations], axis=1)
    table_hi = table.astype(jnp.bfloat16).astype(jnp.float32)
    table = jnp.concatenate([table_hi, table - table_hi], axis=1)
    pos_hi = positions.astype(jnp.bfloat16).astype(jnp.float32)
    pos_hl = jnp.concatenate([pos_hi, positions - pos_hi], axis=1)

    out_a = pl.pallas_call(
        _surface_kernel,
        grid=(_N // _RT,),
        in_specs=[
            pl.BlockSpec((_RT, 3), lambda i: (i, 0)),
            pl.BlockSpec((_RT, 3), lambda i: (i, 0)),
            pl.BlockSpec((3, _M), lambda i: (0, 0)),
            pl.BlockSpec((_M, 20), lambda i: (0, 0)),
        ],
        out_specs=pl.BlockSpec((1, 128), lambda i: (0, 0)),
        out_shape=jax.ShapeDtypeStruct((1, 128), jnp.float32),
        compiler_params=pltpu.CompilerParams(
            dimension_semantics=("arbitrary",)),
    )(tactile_points, tactile_normals, pos_t, table)

    out_b = pl.pallas_call(
        _knn_kernel,
        grid=(_M // _RM,),
        in_specs=[
            pl.BlockSpec((_RM, 3), lambda i: (i, 0)),
            pl.BlockSpec((3, _M), lambda i: (0, 0)),
            pl.BlockSpec((_M, 6), lambda i: (0, 0)),
            pl.BlockSpec((_RM, 1), lambda i: (i, 0)),
            pl.BlockSpec((_RM, 3), lambda i: (i, 0)),
        ],
        out_specs=pl.BlockSpec((1, 128), lambda i: (0, 0)),
        out_shape=jax.ShapeDtypeStruct((1, 128), jnp.float32),
        compiler_params=pltpu.CompilerParams(
            dimension_semantics=("arbitrary",)),
    )(positions, pos_t, pos_hl, opacity, scales)

    surface_loss = out_a[0, 0] / _N
    normal_loss = out_a[0, 1] / _N
    gradient_loss = out_b[0, 0] / (_M * _K * _K)
    opacity_reg = out_b[0, 1] / _M
    scale_reg = out_b[0, 2] / (_M * 3)

    return (_SURFACE_W * surface_loss
            + _NORMAL_W * normal_loss
            + _GRAD_W * gradient_loss
            + _OPACITY_W * opacity_reg
            + _SCALE_W * scale_reg)


# speedup vs baseline: 2.3988x; 1.0522x over previous
import jax
import jax.numpy as jnp
from jax import lax
from jax.experimental import pallas as pl
from jax.experimental.pallas import tpu as pltpu

_N = 8192
_M = 4096

_SURFACE_W = 1.0
_HUBER_DELTA = 0.01
_NORMAL_W = 0.5
_CAUCHY_SIGMA = 0.05
_GRAD_W = 0.1
_GRAD_SIGMA = 0.02
_OPACITY_W = 0.01
_SCALE_W = 0.001
_K = 6

_RT = 512
_RM = 512

_IMAX = 2147483647


def _acc_lanes(out_ref, i, *vals):
    lane = lax.broadcasted_iota(jnp.int32, (1, 128), 1)
    row = jnp.zeros((1, 128), jnp.float32)
    for j, v in enumerate(vals):
        row = row + jnp.where(lane == j, v, 0.0)

    @pl.when(i == 0)
    def _():
        out_ref[...] = jnp.zeros_like(out_ref)

    out_ref[...] += row


def _diff_d2(p, pos_t_ref):
    px, py, pz = p[:, 0:1], p[:, 1:2], p[:, 2:3]
    gx = pos_t_ref[0:1, :]
    gy = pos_t_ref[1:2, :]
    gz = pos_t_ref[2:3, :]
    dx = px - gx
    dy = py - gy
    dz = pz - gz
    return dx * dx + dy * dy + dz * dz


def _surface_kernel(tp_ref, tn_ref, pos_t_ref, table_ref, out_ref):
    i = pl.program_id(0)
    tp = tp_ref[...]
    d2 = _diff_d2(tp, pos_t_ref)

    iota = lax.broadcasted_iota(jnp.int32, d2.shape, 1)
    bits = lax.bitcast_convert_type(d2, jnp.int32)
    packed = (bits & jnp.int32(-4096)) | iota
    mnp = jnp.min(packed, axis=1, keepdims=True)
    sel = packed == mnp
    mn2 = lax.bitcast_convert_type(mnp[:, 0] & jnp.int32(-4096), jnp.float32)
    nearest_dist = jnp.sqrt(jnp.maximum(mn2, 1e-24))

    onehot = sel.astype(jnp.float32)
    ghl = jnp.dot(onehot, table_ref[...],
                  preferred_element_type=jnp.float32)
    g = ghl[:, 0:10] + ghl[:, 10:20]
    nearest_scales = g[:, 0:3]
    near_pos = g[:, 3:6]
    near_rot = g[:, 6:10]

    adaptive = jnp.mean(jnp.exp(nearest_scales), axis=1)
    nd = nearest_dist / (adaptive + 1e-8)
    ax = jnp.abs(nd)
    huber = jnp.where(ax <= _HUBER_DELTA, 0.5 * nd * nd,
                      _HUBER_DELTA * (ax - 0.5 * _HUBER_DELTA))
    surf_sum = jnp.sum(huber)

    qn = near_rot / jnp.maximum(
        jnp.sqrt(jnp.sum(near_rot * near_rot, axis=1, keepdims=True)), 1e-12)
    w, x, y, z = qn[:, 0:1], qn[:, 1:2], qn[:, 2:3], qn[:, 3:4]
    nrm = jnp.concatenate(
        [2 * (x * z + w * y), 2 * (y * z - w * x), 1 - 2 * (x * x + y * y)],
        axis=1)
    to_p = tp - near_pos
    dp = jnp.sum(nrm * to_p, axis=1, keepdims=True)
    nrm = jnp.where(dp < 0, -nrm, nrm)
    nrm = nrm / jnp.maximum(
        jnp.sqrt(jnp.sum(nrm * nrm, axis=1, keepdims=True)), 1e-12)
    dot = jnp.sum(tn_ref[...] * nrm, axis=1)
    ae = 1.0 - jnp.abs(dot)
    sig2 = _CAUCHY_SIGMA * _CAUCHY_SIGMA
    cauchy = -jnp.log(sig2 / (sig2 + ae * ae) + 1e-8)
    normal_sum = jnp.sum(cauchy)

    _acc_lanes(out_ref, i, surf_sum, normal_sum)


def _knn_kernel(pos_ref, pos_t_ref, pos_full_ref, op_ref, sc_ref, out_ref):
    i = pl.program_id(0)
    p = pos_ref[...]
    d2 = _diff_d2(p, pos_t_ref)

    iota = lax.broadcasted_iota(jnp.int32, d2.shape, 1)
    bits = lax.bitcast_convert_type(d2, jnp.int32)
    packed = (bits & jnp.int32(-4096)) | iota
    row_g = i * _RM + lax.broadcasted_iota(jnp.int32, d2.shape, 0)
    packed = jnp.where(iota == row_g, jnp.int32(_IMAX), packed)

    px, py, pz = p[:, 0:1], p[:, 1:2], p[:, 2:3]
    nbx, nby, nbz = [], [], []
    for _ in range(_K):
        mnp = jnp.min(packed, axis=1, keepdims=True)
        sel = packed == mnp
        onehot = sel.astype(jnp.float32)
        nbhl = jnp.dot(onehot, pos_full_ref[...],
                       preferred_element_type=jnp.float32)
        nb = nbhl[:, 0:3] + nbhl[:, 3:6]
        nbx.append(nb[:, 0:1])
        nby.append(nb[:, 1:2])
        nbz.append(nb[:, 2:3])
        packed = jnp.where(sel, jnp.int32(_IMAX), packed)
    nbx = jnp.concatenate(nbx, axis=1)
    nby = jnp.concatenate(nby, axis=1)
    nbz = jnp.concatenate(nbz, axis=1)

    vx = nbx - px
    vy = nby - py
    vz = nbz - pz
    vn = jnp.maximum(jnp.sqrt(vx * vx + vy * vy + vz * vz), 1e-12)
    nvx = vx / vn
    nvy = vy / vn
    nvz = vz / vn

    grad_sum = jnp.float32(0.0)
    for k in range(_K):
        g = (nvx * nvx[:, k:k + 1] + nvy * nvy[:, k:k + 1]
             + nvz * nvz[:, k:k + 1])
        g = jnp.clip(g, -1.0 + 1e-7, 1.0 - 1e-7)
        ang = jnp.arctan2(jnp.sqrt(jnp.maximum(1.0 - g * g, 0.0)), g)
        grad_sum += jnp.sum(jnp.exp(-ang / _GRAD_SIGMA))

    op = op_ref[...]
    op_sum = jnp.sum(op * (1.0 - op))
    scale_sum = jnp.sum(jnp.exp(sc_ref[...]))

    _acc_lanes(out_ref, i, grad_sum, op_sum, scale_sum)


def kernel(tactile_points, tactile_normals, positions, scales, rotations,
           opacity):
    pos_t = positions.T
    table = jnp.concatenate([scales, positions, rotations], axis=1)
    table_hi = table.astype(jnp.bfloat16).astype(jnp.float32)
    table = jnp.concatenate([table_hi, table - table_hi], axis=1)
    pos_hi = positions.astype(jnp.bfloat16).astype(jnp.float32)
    pos_hl = jnp.concatenate([pos_hi, positions - pos_hi], axis=1)

    out_a = pl.pallas_call(
        _surface_kernel,
        grid=(_N // _RT,),
        in_specs=[
            pl.BlockSpec((_RT, 3), lambda i: (i, 0)),
            pl.BlockSpec((_RT, 3), lambda i: (i, 0)),
            pl.BlockSpec((3, _M), lambda i: (0, 0)),
            pl.BlockSpec((_M, 20), lambda i: (0, 0)),
        ],
        out_specs=pl.BlockSpec((1, 128), lambda i: (0, 0)),
        out_shape=jax.ShapeDtypeStruct((1, 128), jnp.float32),
        compiler_params=pltpu.CompilerParams(
            dimension_semantics=("arbitrary",)),
    )(tactile_points, tactile_normals, pos_t, table)

    out_b = pl.pallas_call(
        _knn_kernel,
        grid=(_M // _RM,),
        in_specs=[
            pl.BlockSpec((_RM, 3), lambda i: (i, 0)),
            pl.BlockSpec((3, _M), lambda i: (0, 0)),
            pl.BlockSpec((_M, 6), lambda i: (0, 0)),
            pl.BlockSpec((_RM, 1), lambda i: (i, 0)),
            pl.BlockSpec((_RM, 3), lambda i: (i, 0)),
        ],
        out_specs=pl.BlockSpec((1, 128), lambda i: (0, 0)),
        out_shape=jax.ShapeDtypeStruct((1, 128), jnp.float32),
        compiler_params=pltpu.CompilerParams(
            dimension_semantics=("arbitrary",)),
    )(positions, pos_t, pos_hl, opacity, scales)

    surface_loss = out_a[0, 0] / _N
    normal_loss = out_a[0, 1] / _N
    gradient_loss = out_b[0, 0] / (_M * _K * _K)
    opacity_reg = out_b[0, 1] / _M
    scale_reg = out_b[0, 2] / (_M * 3)

    return (_SURFACE_W * surface_loss
            + _NORMAL_W * normal_loss
            + _GRAD_W * gradient_loss
            + _OPACITY_W * opacity_reg
            + _SCALE_W * scale_reg)


# speedup vs baseline: 2.6205x; 1.0924x over previous
import jax
import jax.numpy as jnp
from jax import lax
from jax.experimental import pallas as pl
from jax.experimental.pallas import tpu as pltpu

_N = 8192
_M = 4096

_SURFACE_W = 1.0
_HUBER_DELTA = 0.01
_NORMAL_W = 0.5
_CAUCHY_SIGMA = 0.05
_GRAD_W = 0.1
_GRAD_SIGMA = 0.02
_OPACITY_W = 0.01
_SCALE_W = 0.001
_K = 6

_RT = 512
_RM = 512

_IMAX = 2147483647


def _acc_lanes(out_ref, i, *vals):
    lane = lax.broadcasted_iota(jnp.int32, (1, 128), 1)
    row = jnp.zeros((1, 128), jnp.float32)
    for j, v in enumerate(vals):
        row = row + jnp.where(lane == j, v, 0.0)

    @pl.when(i == 0)
    def _():
        out_ref[...] = jnp.zeros_like(out_ref)

    out_ref[...] += row


def _diff_d2(p, pos_t_ref):
    pos_t = pos_t_ref[...]
    pad = jnp.zeros((p.shape[0], 5), jnp.float32)
    p8 = jnp.concatenate([p, pad], axis=1)
    pad_t = jnp.zeros((5, pos_t.shape[1]), jnp.float32)
    pos_t8 = jnp.concatenate([pos_t, pad_t], axis=0)
    g = jnp.dot(p8, pos_t8, preferred_element_type=jnp.float32)
    a2 = jnp.sum(p * p, axis=1, keepdims=True)
    gx = pos_t[0:1, :]
    gy = pos_t[1:2, :]
    gz = pos_t[2:3, :]
    b2 = gx * gx + gy * gy + gz * gz
    return jnp.maximum((a2 + b2) - 2.0 * g, 0.0)


def _surface_kernel(tp_ref, tn_ref, pos_t_ref, table_ref, out_ref):
    i = pl.program_id(0)
    tp = tp_ref[...]
    d2 = _diff_d2(tp, pos_t_ref)

    iota = lax.broadcasted_iota(jnp.int32, d2.shape, 1)
    bits = lax.bitcast_convert_type(d2, jnp.int32)
    packed = (bits & jnp.int32(-4096)) | iota
    mnp = jnp.min(packed, axis=1, keepdims=True)
    sel = packed == mnp
    mn2 = lax.bitcast_convert_type(mnp[:, 0] & jnp.int32(-4096), jnp.float32)
    nearest_dist = jnp.sqrt(jnp.maximum(mn2, 1e-24))

    onehot = sel.astype(jnp.float32)
    ghl = jnp.dot(onehot, table_ref[...],
                  preferred_element_type=jnp.float32)
    g = ghl[:, 0:10] + ghl[:, 10:20]
    nearest_scales = g[:, 0:3]
    near_pos = g[:, 3:6]
    near_rot = g[:, 6:10]

    adaptive = jnp.mean(jnp.exp(nearest_scales), axis=1)
    nd = nearest_dist / (adaptive + 1e-8)
    ax = jnp.abs(nd)
    huber = jnp.where(ax <= _HUBER_DELTA, 0.5 * nd * nd,
                      _HUBER_DELTA * (ax - 0.5 * _HUBER_DELTA))
    surf_sum = jnp.sum(huber)

    qn = near_rot / jnp.maximum(
        jnp.sqrt(jnp.sum(near_rot * near_rot, axis=1, keepdims=True)), 1e-12)
    w, x, y, z = qn[:, 0:1], qn[:, 1:2], qn[:, 2:3], qn[:, 3:4]
    nrm = jnp.concatenate(
        [2 * (x * z + w * y), 2 * (y * z - w * x), 1 - 2 * (x * x + y * y)],
        axis=1)
    to_p = tp - near_pos
    dp = jnp.sum(nrm * to_p, axis=1, keepdims=True)
    nrm = jnp.where(dp < 0, -nrm, nrm)
    nrm = nrm / jnp.maximum(
        jnp.sqrt(jnp.sum(nrm * nrm, axis=1, keepdims=True)), 1e-12)
    dot = jnp.sum(tn_ref[...] * nrm, axis=1)
    ae = 1.0 - jnp.abs(dot)
    sig2 = _CAUCHY_SIGMA * _CAUCHY_SIGMA
    cauchy = -jnp.log(sig2 / (sig2 + ae * ae) + 1e-8)
    normal_sum = jnp.sum(cauchy)

    _acc_lanes(out_ref, i, surf_sum, normal_sum)


def _knn_kernel(pos_ref, pos_t_ref, pos_full_ref, op_ref, sc_ref, out_ref):
    i = pl.program_id(0)
    p = pos_ref[...]
    d2 = _diff_d2(p, pos_t_ref)

    iota = lax.broadcasted_iota(jnp.int32, d2.shape, 1)
    bits = lax.bitcast_convert_type(d2, jnp.int32)
    packed = (bits & jnp.int32(-4096)) | iota
    row_g = i * _RM + lax.broadcasted_iota(jnp.int32, d2.shape, 0)
    packed = jnp.where(iota == row_g, jnp.int32(_IMAX), packed)

    px, py, pz = p[:, 0:1], p[:, 1:2], p[:, 2:3]
    nbx, nby, nbz = [], [], []
    for _ in range(_K):
        mnp = jnp.min(packed, axis=1, keepdims=True)
        sel = packed == mnp
        onehot = sel.astype(jnp.float32)
        nbhl = jnp.dot(onehot, pos_full_ref[...],
                       preferred_element_type=jnp.float32)
        nb = nbhl[:, 0:3] + nbhl[:, 3:6]
        nbx.append(nb[:, 0:1])
        nby.append(nb[:, 1:2])
        nbz.append(nb[:, 2:3])
        packed = jnp.where(sel, jnp.int32(_IMAX), packed)
    nbx = jnp.concatenate(nbx, axis=1)
    nby = jnp.concatenate(nby, axis=1)
    nbz = jnp.concatenate(nbz, axis=1)

    vx = nbx - px
    vy = nby - py
    vz = nbz - pz
    vn = jnp.maximum(jnp.sqrt(vx * vx + vy * vy + vz * vz), 1e-12)
    nvx = vx / vn
    nvy = vy / vn
    nvz = vz / vn

    grad_sum = jnp.float32(0.0)
    for k in range(_K):
        g = (nvx * nvx[:, k:k + 1] + nvy * nvy[:, k:k + 1]
             + nvz * nvz[:, k:k + 1])
        g = jnp.clip(g, -1.0 + 1e-7, 1.0 - 1e-7)
        ang = jnp.arctan2(jnp.sqrt(jnp.maximum(1.0 - g * g, 0.0)), g)
        grad_sum += jnp.sum(jnp.exp(-ang / _GRAD_SIGMA))

    op = op_ref[...]
    op_sum = jnp.sum(op * (1.0 - op))
    scale_sum = jnp.sum(jnp.exp(sc_ref[...]))

    _acc_lanes(out_ref, i, grad_sum, op_sum, scale_sum)


def kernel(tactile_points, tactile_normals, positions, scales, rotations,
           opacity):
    pos_t = positions.T
    table = jnp.concatenate([scales, positions, rotations], axis=1)
    table_hi = table.astype(jnp.bfloat16).astype(jnp.float32)
    table = jnp.concatenate([table_hi, table - table_hi], axis=1)
    pos_hi = positions.astype(jnp.bfloat16).astype(jnp.float32)
    pos_hl = jnp.concatenate([pos_hi, positions - pos_hi], axis=1)

    out_a = pl.pallas_call(
        _surface_kernel,
        grid=(_N // _RT,),
        in_specs=[
            pl.BlockSpec((_RT, 3), lambda i: (i, 0)),
            pl.BlockSpec((_RT, 3), lambda i: (i, 0)),
            pl.BlockSpec((3, _M), lambda i: (0, 0)),
            pl.BlockSpec((_M, 20), lambda i: (0, 0)),
        ],
        out_specs=pl.BlockSpec((1, 128), lambda i: (0, 0)),
        out_shape=jax.ShapeDtypeStruct((1, 128), jnp.float32),
        compiler_params=pltpu.CompilerParams(
            dimension_semantics=("arbitrary",)),
    )(tactile_points, tactile_normals, pos_t, table)

    out_b = pl.pallas_call(
        _knn_kernel,
        grid=(_M // _RM,),
        in_specs=[
            pl.BlockSpec((_RM, 3), lambda i: (i, 0)),
            pl.BlockSpec((3, _M), lambda i: (0, 0)),
            pl.BlockSpec((_M, 6), lambda i: (0, 0)),
            pl.BlockSpec((_RM, 1), lambda i: (i, 0)),
            pl.BlockSpec((_RM, 3), lambda i: (i, 0)),
        ],
        out_specs=pl.BlockSpec((1, 128), lambda i: (0, 0)),
        out_shape=jax.ShapeDtypeStruct((1, 128), jnp.float32),
        compiler_params=pltpu.CompilerParams(
            dimension_semantics=("arbitrary",)),
    )(positions, pos_t, pos_hl, opacity, scales)

    surface_loss = out_a[0, 0] / _N
    normal_loss = out_a[0, 1] / _N
    gradient_loss = out_b[0, 0] / (_M * _K * _K)
    opacity_reg = out_b[0, 1] / _M
    scale_reg = out_b[0, 2] / (_M * 3)

    return (_SURFACE_W * surface_loss
            + _NORMAL_W * normal_loss
            + _GRAD_W * gradient_loss
            + _OPACITY_W * opacity_reg
            + _SCALE_W * scale_reg)
